# baseline (speedup 1.0000x reference)
import functools

import jax
import jax.numpy as jnp
from jax.experimental import pallas as pl
from jax.experimental.pallas import tpu as pltpu

SEQ_LEN = 2048
START_TOK = 0
END_TOK = 2
PAD_TOK = 1


def _trim_budgets(L1, L2, budget):
    if L1 + L2 <= budget:
        return L1, L2
    k1 = min(L1, max((budget + 1) // 2, budget - L2))
    k2 = min(L2, max(budget // 2, budget - L1))
    return max(k1, 0), max(k2, 0)


@functools.cache
def _build_packer(B, L1, L2):
    budget = SEQ_LEN - 4
    k1, k2 = _trim_budgets(L1, L2, budget)
    half = SEQ_LEN // 2
    assert k1 == half - 2 and k2 == half - 2
    w1 = min(L1, -(-k1 // 128) * 128)
    w2 = min(L2, -(-k2 // 128) * 128)

    def body(s1_hbm, s2_hbm, o_hbm, a_v, b_v, o1_v, o2_v,
             sem_a, sem_b, sem_o1, sem_o2):
        cpa = pltpu.make_async_copy(s1_hbm.at[:, pl.ds(0, w1)], a_v, sem_a)
        cpb = pltpu.make_async_copy(s2_hbm.at[:, pl.ds(0, w2)], b_v, sem_b)
        cpa.start()
        cpb.start()

        start = jnp.full((B, 1), START_TOK, jnp.int32)
        split = jnp.full((B, 1), END_TOK, jnp.int32)

        cpa.wait()
        o1_v[...] = jnp.concatenate([start, a_v[:, :k1], split], axis=1)
        cpo1 = pltpu.make_async_copy(o1_v, o_hbm.at[:, pl.ds(0, half)], sem_o1)
        cpo1.start()

        cpb.wait()
        o2_v[...] = jnp.concatenate([split, b_v[:, :k2], split], axis=1)
        cpo2 = pltpu.make_async_copy(o2_v, o_hbm.at[:, pl.ds(half, half)],
                                     sem_o2)
        cpo2.start()

        cpo1.wait()
        cpo2.wait()

    return pl.pallas_call(
        body,
        in_specs=[
            pl.BlockSpec(memory_space=pltpu.MemorySpace.HBM),
            pl.BlockSpec(memory_space=pltpu.MemorySpace.HBM),
        ],
        out_specs=pl.BlockSpec(memory_space=pltpu.MemorySpace.HBM),
        out_shape=jax.ShapeDtypeStruct((B, SEQ_LEN), jnp.int32),
        scratch_shapes=[
            pltpu.VMEM((B, w1), jnp.int32),
            pltpu.VMEM((B, w2), jnp.int32),
            pltpu.VMEM((B, half), jnp.int32),
            pltpu.VMEM((B, half), jnp.int32),
            pltpu.SemaphoreType.DMA,
            pltpu.SemaphoreType.DMA,
            pltpu.SemaphoreType.DMA,
            pltpu.SemaphoreType.DMA,
        ],
    )


def kernel(segment_1, segment_2):
    B, L1 = segment_1.shape
    L2 = segment_2.shape[1]
    return _build_packer(B, L1, L2)(segment_1, segment_2)
